# baseline (speedup 1.0000x reference)
import jax
import jax.numpy as jnp
from jax import lax
from jax.experimental import pallas as pl
from jax.experimental.pallas import tpu as pltpu
from jax.experimental.pallas import tpu_sc as plsc

B = 1024
INPUT_DIM = 512
NB_VALUES = 256
NB_EMB = 100000
EMB_DIM = 128
HIDDEN = 1024
N_ACTIONS = 18

NC = 2
NS = 16
NW = NC * NS
BPW = B // NW
IDX_PER_W = BPW * INPUT_DIM
CHUNK = 128
CHUNKS_PER_ROW = INPUT_DIM // CHUNK
NCHUNKS = IDX_PER_W // CHUNK
NBUF = 8
MCOLS = 64
LANES = 16
KREG = EMB_DIM // LANES
KPACK = EMB_DIM // (2 * LANES)

def _permute_w1(w1):
    return w1.reshape(HIDDEN, 2, 4, 16).transpose(0, 2, 1, 3).reshape(
        HIDDEN, EMB_DIM)


MASK_HI = -65536


def _unpack(w):
    lo = lax.bitcast_convert_type(lax.shift_left(w, 16), jnp.float32)
    hi = lax.bitcast_convert_type(lax.bitwise_and(w, MASK_HI), jnp.float32)
    return lo, hi


def _bag_body(x_hbm, map_hbm, emb_hbm, out_hbm,
              xv, mslice, rid, ebuf, row0, r0f, obuf, sem_emb):
    wid = lax.axis_index("s") * NC + lax.axis_index("c")
    base = wid * BPW

    pltpu.sync_copy(x_hbm.at[pl.ds(base, BPW)], xv)
    pltpu.sync_copy(emb_hbm.at[pl.ds(0, 8)], row0)
    for k in range(KPACK):
        lo, hi = _unpack(row0[0, pl.ds(k * LANES, LANES)])
        r0f[0, pl.ds((2 * k) * LANES, LANES)] = lo
        r0f[0, pl.ds((2 * k + 1) * LANES, LANES)] = hi

    jota = lax.iota(jnp.int32, LANES) * NB_VALUES

    def piece_body(p, _):
        pltpu.sync_copy(map_hbm.at[pl.ds(p * 2 * MCOLS, 2 * MCOLS)], mslice)

        def row_rbody(r, _):
            for cg in range(MCOLS // LANES):
                x = xv[r, pl.ds(p * MCOLS + cg * LANES, LANES)]
                flat = x + jota + cg * (LANES * NB_VALUES)
                v = plsc.load_gather(
                    mslice, [lax.shift_right_logical(flat, 7), flat & 127])
                rid[pl.ds(r * INPUT_DIM + p * MCOLS + cg * LANES, LANES)] = v
            return 0

        lax.fori_loop(0, BPW, row_rbody, 0)
        return 0

    lax.fori_loop(0, INPUT_DIM // MCOLS, piece_body, 0)

    def fire(c, buf):
        pltpu.async_copy(
            emb_hbm.at[rid.at[pl.ds(c * CHUNK, CHUNK)]],
            ebuf.at[buf],
            sem_emb,
        )

    def wait_one():
        pltpu.make_async_copy(emb_hbm.at[pl.ds(0, CHUNK)], ebuf.at[0],
                              sem_emb).wait()

    for i in range(NBUF - 1):
        fire(i, i)

    def row_body(b_loc, _):
        def cnt_body(q, cv):
            v = rid[pl.ds(b_loc * INPUT_DIM + q * LANES, LANES)]
            return cv + plsc.all_reduce_population_count(v == 0)

        cnt = lax.fori_loop(0, INPUT_DIM // LANES, cnt_body,
                            jnp.zeros((LANES,), jnp.int32)).astype(jnp.float32)

        acc = tuple(-cnt * r0f[0, pl.ds(k * LANES, LANES)]
                    for k in range(KREG))
        bufb = (b_loc % 2) * CHUNKS_PER_ROW
        for q in range(CHUNKS_PER_ROW):
            c = b_loc * CHUNKS_PER_ROW + q
            nxt = c + NBUF - 1

            @pl.when(nxt < NCHUNKS)
            def _():
                fire(nxt, lax.rem(nxt, NBUF))

            wait_one()

            def acc_body(rr, a):
                a = list(a)
                for k in range(KPACK):
                    w = ebuf[bufb + q, rr, pl.ds(k * LANES, LANES)]
                    lo, hi = _unpack(w)
                    a[2 * k] = a[2 * k] + lo
                    a[2 * k + 1] = a[2 * k + 1] + hi
                return tuple(a)

            acc = lax.fori_loop(0, CHUNK, acc_body, acc)

        for k in range(KREG):
            obuf[b_loc, pl.ds(k * LANES, LANES)] = acc[k]
        return 0

    lax.fori_loop(0, BPW, row_body, 0)
    pltpu.sync_copy(obuf, out_hbm.at[pl.ds(base, BPW)])


PACK_ROWS = 5000


def _pack_body(x, out):
    xu = lax.bitcast_convert_type(x[...], jnp.uint32)
    r16 = (xu + jnp.uint32(0x7FFF) + ((xu >> 16) & jnp.uint32(1))) >> 16
    rot = pltpu.roll(r16, 64, axis=1)
    out[...] = lax.bitcast_convert_type(
        (r16 | (rot << 16))[:, : EMB_DIM // 2], jnp.int32)


@jax.jit
def _pack(emb):
    return pl.pallas_call(
        _pack_body,
        grid=(NB_EMB // PACK_ROWS,),
        in_specs=[pl.BlockSpec((PACK_ROWS, EMB_DIM), lambda i: (i, 0))],
        out_specs=pl.BlockSpec((PACK_ROWS, EMB_DIM // 2), lambda i: (i, 0)),
        out_shape=jax.ShapeDtypeStruct((NB_EMB, EMB_DIM // 2), jnp.int32),
    )(emb)


@jax.jit
def _bag(x, mapping, emb_packed):
    map128 = mapping.reshape(-1, 128)
    mesh = plsc.VectorSubcoreMesh(core_axis_name="c", subcore_axis_name="s",
                                  num_cores=NC, num_subcores=NS)
    return pl.kernel(
        _bag_body,
        out_type=jax.ShapeDtypeStruct((B, EMB_DIM), jnp.float32),
        mesh=mesh,
        compiler_params=pltpu.CompilerParams(needs_layout_passes=False,
                                             use_tc_tiling_on_sc=False),
        scratch_types=[
            pltpu.VMEM((BPW, INPUT_DIM), jnp.int32),
            pltpu.VMEM((2 * MCOLS, 128), jnp.int32),
            pltpu.VMEM((IDX_PER_W,), jnp.int32),
            pltpu.VMEM((NBUF, CHUNK, EMB_DIM // 2), jnp.int32),
            pltpu.VMEM((8, EMB_DIM // 2), jnp.int32),
            pltpu.VMEM((1, EMB_DIM), jnp.float32),
            pltpu.VMEM((BPW, EMB_DIM), jnp.float32),
            pltpu.SemaphoreType.DMA,
        ],
    )(x, map128, emb_packed)


def _mlp_body(bag, w1, b1, w2, b2, w3, b3, out):
    dn = (((1,), (1,)), ((), ()))
    h = jnp.maximum(bag[...], 0.0)
    h = lax.dot_general(h, w1[...], dn, preferred_element_type=jnp.float32)
    h = jnp.maximum(h + b1[...], 0.0)
    h = lax.dot_general(h, w2[...], dn, preferred_element_type=jnp.float32)
    h = jnp.maximum(h + b2[...], 0.0)
    h = lax.dot_general(h, w3[...], dn, preferred_element_type=jnp.float32)
    out[...] = jnp.maximum(h + b3[...], 0.0)


@jax.jit
def _mlp(bag, w1, b1, w2, b2, w3, b3):
    return pl.pallas_call(
        _mlp_body,
        out_shape=jax.ShapeDtypeStruct((B, N_ACTIONS), jnp.float32),
    )(bag, _permute_w1(w1), b1.reshape(1, HIDDEN), w2,
      b2.reshape(1, HIDDEN), w3, b3.reshape(1, N_ACTIONS))


def kernel(X, mapping_filtered, emb_weight, W1, b1, W2, b2, W3, b3):
    bag = _bag(X, mapping_filtered, _pack(emb_weight))
    return _mlp(bag, W1, b1, W2, b2, W3, b3)


# speedup vs baseline: 1.3673x; 1.3673x over previous
import jax
import jax.numpy as jnp
from jax import lax
from jax.experimental import pallas as pl
from jax.experimental.pallas import tpu as pltpu
from jax.experimental.pallas import tpu_sc as plsc

B = 1024
INPUT_DIM = 512
NB_VALUES = 256
NB_EMB = 100000
EMB_DIM = 128
HIDDEN = 1024
N_ACTIONS = 18

NC = 2
NS = 16
NW = NC * NS
BPW = B // NW
IDX_PER_W = BPW * INPUT_DIM
CHUNK = 64
CHUNKS_PER_ROW = INPUT_DIM // CHUNK
NCHUNKS = IDX_PER_W // CHUNK
NBUF = 8
RAHEAD = 8
LANES = 16
KREG = EMB_DIM // LANES


def _bag_body(x_hbm, map_hbm, emb_hbm, out_hbm,
              xv, fidx, rid, ebuf, row0, obuf, sem_map, sem_emb):
    wid = lax.axis_index("s") * NC + lax.axis_index("c")
    base = wid * BPW

    pltpu.sync_copy(x_hbm.at[pl.ds(base, BPW)], xv)
    pltpu.sync_copy(emb_hbm.at[pl.ds(0, 1)], row0)

    jota = lax.iota(jnp.int32, LANES) * NB_VALUES

    def fidx_body(g, _):
        r = g // (INPUT_DIM // LANES)
        cc = g % (INPUT_DIM // LANES)
        x = xv[r, pl.ds(cc * LANES, LANES)]
        fidx[pl.ds(g * LANES, LANES)] = x + jota + cc * (LANES * NB_VALUES)
        return 0

    lax.fori_loop(0, IDX_PER_W // LANES, fidx_body, 0)

    def remap_fire_group(g):
        def one(c, _):
            pltpu.async_copy(
                map_hbm.at[fidx.at[pl.ds(c * CHUNK, CHUNK)]],
                rid.at[pl.ds(c * CHUNK, CHUNK)],
                sem_map,
            )
            return 0

        lax.fori_loop(g * CHUNKS_PER_ROW, (g + 1) * CHUNKS_PER_ROW, one, 0)

    def remap_wait_group():
        pltpu.make_async_copy(map_hbm.at[pl.ds(0, INPUT_DIM)],
                              rid.at[pl.ds(0, INPUT_DIM)], sem_map).wait()

    def prologue_fire(g, _):
        remap_fire_group(g)
        return 0

    lax.fori_loop(0, RAHEAD, prologue_fire, 0)
    remap_wait_group()

    def fire(c, buf):
        pltpu.async_copy(
            emb_hbm.at[rid.at[pl.ds(c * CHUNK, CHUNK)]],
            ebuf.at[buf],
            sem_emb,
        )

    def wait_one():
        pltpu.make_async_copy(emb_hbm.at[pl.ds(0, CHUNK)], ebuf.at[0],
                              sem_emb).wait()

    for i in range(NBUF - 1):
        fire(i, i)

    def row_body(b_loc, _):
        @pl.when(b_loc + RAHEAD < BPW)
        def _():
            remap_fire_group(b_loc + RAHEAD)

        @pl.when(b_loc < BPW - 1)
        def _():
            remap_wait_group()

        def cnt_body(q, cv):
            v = rid[pl.ds(b_loc * INPUT_DIM + q * LANES, LANES)]
            return cv + plsc.all_reduce_population_count(v == 0)

        cnt = lax.fori_loop(0, INPUT_DIM // LANES, cnt_body,
                            jnp.zeros((LANES,), jnp.int32)).astype(jnp.float32)

        acc = tuple(-cnt * row0[0, pl.ds(k * LANES, LANES)]
                    for k in range(KREG))
        for q in range(CHUNKS_PER_ROW):
            c = b_loc * CHUNKS_PER_ROW + q
            nxt = c + NBUF - 1

            @pl.when(nxt < NCHUNKS)
            def _():
                fire(nxt, (q + NBUF - 1) % NBUF)

            wait_one()

            def acc_body(rr, a):
                return tuple(a[k] + ebuf[q, rr, pl.ds(k * LANES, LANES)]
                             for k in range(KREG))

            acc = lax.fori_loop(0, CHUNK, acc_body, acc)

        for k in range(KREG):
            obuf[b_loc, pl.ds(k * LANES, LANES)] = acc[k]
        return 0

    lax.fori_loop(0, BPW, row_body, 0)
    pltpu.sync_copy(obuf, out_hbm.at[pl.ds(base, BPW)])


@jax.jit
def _bag(x, map_flat, emb):
    mesh = plsc.VectorSubcoreMesh(core_axis_name="c", subcore_axis_name="s",
                                  num_cores=NC, num_subcores=NS)
    return pl.kernel(
        _bag_body,
        out_type=jax.ShapeDtypeStruct((B, EMB_DIM), jnp.float32),
        mesh=mesh,
        compiler_params=pltpu.CompilerParams(needs_layout_passes=False),
        scratch_types=[
            pltpu.VMEM((BPW, INPUT_DIM), jnp.int32),
            pltpu.VMEM((IDX_PER_W,), jnp.int32),
            pltpu.VMEM((IDX_PER_W,), jnp.int32),
            pltpu.VMEM((NBUF, CHUNK, EMB_DIM), jnp.float32),
            pltpu.VMEM((1, EMB_DIM), jnp.float32),
            pltpu.VMEM((BPW, EMB_DIM), jnp.float32),
            pltpu.SemaphoreType.DMA,
            pltpu.SemaphoreType.DMA,
        ],
    )(x, map_flat, emb)


def _mlp_body(bag, w1, b1, w2, b2, w3, b3, out):
    dn = (((1,), (1,)), ((), ()))
    h = jnp.maximum(bag[...], 0.0)
    h = lax.dot_general(h, w1[...], dn, preferred_element_type=jnp.float32)
    h = jnp.maximum(h + b1[...], 0.0)
    h = lax.dot_general(h, w2[...], dn, preferred_element_type=jnp.float32)
    h = jnp.maximum(h + b2[...], 0.0)
    h = lax.dot_general(h, w3[...], dn, preferred_element_type=jnp.float32)
    out[...] = jnp.maximum(h + b3[...], 0.0)


@jax.jit
def _mlp(bag, w1, b1, w2, b2, w3, b3):
    return pl.pallas_call(
        _mlp_body,
        out_shape=jax.ShapeDtypeStruct((B, N_ACTIONS), jnp.float32),
    )(bag, w1, b1, w2, b2, w3, b3)


def kernel(X, mapping_filtered, emb_weight, W1, b1, W2, b2, W3, b3):
    map_flat = mapping_filtered.reshape(-1)
    bag = _bag(X, map_flat, emb_weight)
    return _mlp(bag, W1, b1.reshape(1, HIDDEN), W2, b2.reshape(1, HIDDEN),
                W3, b3.reshape(1, N_ACTIONS))


# speedup vs baseline: 1.3763x; 1.0066x over previous
import jax
import jax.numpy as jnp
from jax import lax
from jax.experimental import pallas as pl
from jax.experimental.pallas import tpu as pltpu
from jax.experimental.pallas import tpu_sc as plsc

B = 1024
INPUT_DIM = 512
NB_VALUES = 256
NB_EMB = 100000
EMB_DIM = 128
HIDDEN = 1024
N_ACTIONS = 18

NC = 2
NS = 16
NW = NC * NS
BPW = B // NW
IDX_PER_W = BPW * INPUT_DIM
CHUNK = 128
CHUNKS_PER_ROW = INPUT_DIM // CHUNK
NCHUNKS = IDX_PER_W // CHUNK
NBUF = 4
RAHEAD = 8
LANES = 16
KREG = EMB_DIM // LANES


def _bag_body(x_hbm, map_hbm, emb_hbm, out_hbm,
              xv, fidx, rid, ebuf, row0, obuf, sem_map, sem_emb):
    wid = lax.axis_index("s") * NC + lax.axis_index("c")
    base = wid * BPW

    pltpu.sync_copy(x_hbm.at[pl.ds(base, BPW)], xv)
    pltpu.sync_copy(emb_hbm.at[pl.ds(0, 1)], row0)

    jota = lax.iota(jnp.int32, LANES) * NB_VALUES

    def fidx_body(g, _):
        r = g // (INPUT_DIM // LANES)
        cc = g % (INPUT_DIM // LANES)
        x = xv[r, pl.ds(cc * LANES, LANES)]
        fidx[pl.ds(g * LANES, LANES)] = x + jota + cc * (LANES * NB_VALUES)
        return 0

    lax.fori_loop(0, IDX_PER_W // LANES, fidx_body, 0)

    def remap_fire_group(g):
        def one(c, _):
            pltpu.async_copy(
                map_hbm.at[fidx.at[pl.ds(c * CHUNK, CHUNK)]],
                rid.at[pl.ds(c * CHUNK, CHUNK)],
                sem_map,
            )
            return 0

        lax.fori_loop(g * CHUNKS_PER_ROW, (g + 1) * CHUNKS_PER_ROW, one, 0)

    def remap_wait_group():
        pltpu.make_async_copy(map_hbm.at[pl.ds(0, INPUT_DIM)],
                              rid.at[pl.ds(0, INPUT_DIM)], sem_map).wait()

    def prologue_fire(g, _):
        remap_fire_group(g)
        return 0

    lax.fori_loop(0, RAHEAD, prologue_fire, 0)
    remap_wait_group()

    def fire(c, buf):
        pltpu.async_copy(
            emb_hbm.at[rid.at[pl.ds(c * CHUNK, CHUNK)]],
            ebuf.at[buf],
            sem_emb,
        )

    def wait_one():
        pltpu.make_async_copy(emb_hbm.at[pl.ds(0, CHUNK)], ebuf.at[0],
                              sem_emb).wait()

    for i in range(NBUF - 1):
        fire(i, i)

    def row_body(b_loc, _):
        @pl.when(b_loc + RAHEAD < BPW)
        def _():
            remap_fire_group(b_loc + RAHEAD)

        @pl.when(b_loc < BPW - 1)
        def _():
            remap_wait_group()

        def cnt_body(q, cv):
            v = rid[pl.ds(b_loc * INPUT_DIM + q * LANES, LANES)]
            return cv + plsc.all_reduce_population_count(v == 0)

        cnt = lax.fori_loop(0, INPUT_DIM // LANES, cnt_body,
                            jnp.zeros((LANES,), jnp.int32)).astype(jnp.float32)

        acc = tuple(-cnt * row0[0, pl.ds(k * LANES, LANES)]
                    for k in range(KREG))
        for q in range(CHUNKS_PER_ROW):
            c = b_loc * CHUNKS_PER_ROW + q
            nxt = c + NBUF - 1

            @pl.when(nxt < NCHUNKS)
            def _():
                fire(nxt, (q + NBUF - 1) % NBUF)

            wait_one()

            def acc_body(rr, a):
                return tuple(a[k] + ebuf[q, rr, pl.ds(k * LANES, LANES)]
                             for k in range(KREG))

            acc = lax.fori_loop(0, CHUNK, acc_body, acc)

        for k in range(KREG):
            obuf[b_loc, pl.ds(k * LANES, LANES)] = acc[k]
        return 0

    lax.fori_loop(0, BPW, row_body, 0)
    pltpu.sync_copy(obuf, out_hbm.at[pl.ds(base, BPW)])


@jax.jit
def _bag(x, map_flat, emb):
    mesh = plsc.VectorSubcoreMesh(core_axis_name="c", subcore_axis_name="s",
                                  num_cores=NC, num_subcores=NS)
    return pl.kernel(
        _bag_body,
        out_type=jax.ShapeDtypeStruct((B, EMB_DIM), jnp.float32),
        mesh=mesh,
        compiler_params=pltpu.CompilerParams(needs_layout_passes=False),
        scratch_types=[
            pltpu.VMEM((BPW, INPUT_DIM), jnp.int32),
            pltpu.VMEM((IDX_PER_W,), jnp.int32),
            pltpu.VMEM((IDX_PER_W,), jnp.int32),
            pltpu.VMEM((NBUF, CHUNK, EMB_DIM), jnp.float32),
            pltpu.VMEM((1, EMB_DIM), jnp.float32),
            pltpu.VMEM((BPW, EMB_DIM), jnp.float32),
            pltpu.SemaphoreType.DMA,
            pltpu.SemaphoreType.DMA,
        ],
    )(x, map_flat, emb)


def _mlp_body(bag, w1, b1, w2, b2, w3, b3, out):
    dn = (((1,), (1,)), ((), ()))
    h = jnp.maximum(bag[...], 0.0)
    h = lax.dot_general(h, w1[...], dn, preferred_element_type=jnp.float32)
    h = jnp.maximum(h + b1[...], 0.0)
    h = lax.dot_general(h, w2[...], dn, preferred_element_type=jnp.float32)
    h = jnp.maximum(h + b2[...], 0.0)
    h = lax.dot_general(h, w3[...], dn, preferred_element_type=jnp.float32)
    out[...] = jnp.maximum(h + b3[...], 0.0)


@jax.jit
def _mlp(bag, w1, b1, w2, b2, w3, b3):
    return pl.pallas_call(
        _mlp_body,
        out_shape=jax.ShapeDtypeStruct((B, N_ACTIONS), jnp.float32),
    )(bag, w1, b1, w2, b2, w3, b3)


def kernel(X, mapping_filtered, emb_weight, W1, b1, W2, b2, W3, b3):
    map_flat = mapping_filtered.reshape(-1)
    bag = _bag(X, map_flat, emb_weight)
    return _mlp(bag, W1, b1.reshape(1, HIDDEN), W2, b2.reshape(1, HIDDEN),
                W3, b3.reshape(1, N_ACTIONS))
